# Initial kernel scaffold; baseline (speedup 1.0000x reference)
import jax
import jax.numpy as jnp
from jax.experimental import pallas as pl


def kernel(x, edge_index, W1, b1, Wc, bc):
    raise NotImplementedError("write your pallas kernel here")


# speedup vs baseline: 232.4421x; 232.4421x over previous
import jax
import jax.numpy as jnp
import numpy as np
from jax import lax
from jax.experimental import pallas as pl
from jax.experimental.pallas import tpu as pltpu
from jax.experimental.pallas import tpu_sc as plsc

_N = 100000
_E = 1600000
_DOUT = 40
_QC = 10
_QP = 16
_MU = 0.1
_EPS = 1e-8

_LANE = 128
_CROWS = 8
_CHUNK = _LANE * _CROWS

_PT = 100352
_NCHUNK = _PT // _CHUNK
_EPAD = 16 * _PT
_EROWS = _EPAD // _LANE

_PT_DEG = _EPAD // 32
_NCHUNK_DEG = _PT_DEG // _CHUNK

_PADR = 96
_NR = 100096
_RPT = _NR // 16

_ZD = 368
_ZB = 368

_BR = 2000
_GRID = _N // _BR

_ALPHA_POS = 1.0 / (1.0 + _MU)
_Z = np.int32(0)


def _mesh():
    return plsc.VectorSubcoreMesh(core_axis_name="c", subcore_axis_name="s")


def _mm_body(x_ref, w1t_ref, b1_ref, wct_ref, bc_ref, out_ref):
    h = jnp.maximum(
        jnp.dot(x_ref[...], w1t_ref[...], preferred_element_type=jnp.float32)
        + b1_ref[...], 0.0)
    out_ref[...] = (
        jnp.dot(h, wct_ref[...], preferred_element_type=jnp.float32)
        + bc_ref[...])


def _compute_h(x, w1t, b1, wct, bc):
    return pl.pallas_call(
        _mm_body,
        grid=(_GRID,),
        in_specs=[
            pl.BlockSpec((_BR, 128), lambda i: (i, _Z)),
            pl.BlockSpec((128, 16), lambda i: (_Z, _Z)),
            pl.BlockSpec((1, 16), lambda i: (_Z, _Z)),
            pl.BlockSpec((16, _DOUT), lambda i: (_Z, _Z)),
            pl.BlockSpec((1, _DOUT), lambda i: (_Z, _Z)),
        ],
        out_specs=pl.BlockSpec((_BR, _DOUT), lambda i: (i, _Z)),
        out_shape=jax.ShapeDtypeStruct((_N, _DOUT), jnp.float32),
    )(x, w1t, b1, wct, bc)


def _deg_body(dst2, zflat, d_out, didx_v, ones_v, zd_v, deg_sh, sem):
    del sem
    c = lax.axis_index("c")
    s = lax.axis_index("s")
    w = c * 16 + s
    pltpu.sync_copy(zflat, zd_v)
    for k in range(_RPT // _ZD):
        pltpu.sync_copy(zd_v, deg_sh.at[pl.ds(s * _RPT + k * _ZD, _ZD)])
    for i in range(8):
        ones_v[pl.ds(i * 16, 16)] = jnp.ones((16,), jnp.float32)
    plsc.subcore_barrier()

    rbase = w * jnp.int32(_PT_DEG // _LANE)

    @pl.loop(jnp.int32(0), jnp.int32(_NCHUNK_DEG))
    def _(i):
        r = rbase + i * jnp.int32(_CROWS)
        pltpu.sync_copy(dst2.at[pl.ds(r, _CROWS)], didx_v)
        for j in range(_CROWS):
            pltpu.sync_copy(ones_v, deg_sh.at[didx_v.at[jnp.int32(j)]],
                            add=True)
    plsc.subcore_barrier()

    for k in range(_RPT // _ZD):
        pltpu.sync_copy(deg_sh.at[pl.ds(s * _RPT + k * _ZD, _ZD)], zd_v)
        pltpu.sync_copy(zd_v, d_out.at[c, pl.ds(s * _RPT + k * _ZD, _ZD)])


def _compute_deg(dst2, zflat):
    kern = pl.kernel(
        _deg_body,
        out_type=jax.ShapeDtypeStruct((2, _NR), jnp.float32),
        mesh=_mesh(),
        scratch_types=[
            pltpu.VMEM((_CROWS, _LANE), jnp.int32),
            pltpu.VMEM((_LANE,), jnp.float32),
            pltpu.VMEM((_ZD,), jnp.float32),
            pltpu.VMEM_SHARED((_NR,), jnp.float32),
            pltpu.SemaphoreType.DMA,
        ],
        compiler_params=pltpu.CompilerParams(use_tc_tiling_on_sc=False),
    )
    return kern(dst2, zflat)


def _prep_body(d0_ref, d1_ref, h_ref, uq0_ref, uq1_ref, uq2_ref, uq3_ref,
               bh_ref, dbh_ref, s_ref, sd_ref):
    deg = d0_ref[...] + d1_ref[...]
    pos = deg > 0
    dinv = jnp.where(pos, lax.rsqrt(jnp.maximum(deg, _EPS)), 0.0)
    alpha = jnp.where(pos, jnp.float32(_ALPHA_POS), jnp.float32(1.0 / _MU))
    beta = _MU * alpha
    h = h_ref[...]
    u1 = h * dinv
    zp = jnp.zeros((u1.shape[0], _QP - _QC), jnp.float32)
    uq0_ref[...] = jnp.concatenate([u1[:, 0 * _QC:1 * _QC], zp], axis=1)
    uq1_ref[...] = jnp.concatenate([u1[:, 1 * _QC:2 * _QC], zp], axis=1)
    uq2_ref[...] = jnp.concatenate([u1[:, 2 * _QC:3 * _QC], zp], axis=1)
    uq3_ref[...] = jnp.concatenate([u1[:, 3 * _QC:4 * _QC], zp], axis=1)
    bh = beta * h
    bh_ref[...] = bh
    dbh_ref[...] = dinv * bh
    s_ref[...] = alpha * dinv
    sd_ref[...] = alpha * dinv * dinv


def _prepare(d0, d1, h):
    qspec = pl.BlockSpec((_BR, _QP), lambda i: (i, _Z))
    fspec = pl.BlockSpec((_BR, _DOUT), lambda i: (i, _Z))
    cspec = pl.BlockSpec((_BR, 1), lambda i: (i, _Z))
    qshape = jax.ShapeDtypeStruct((_N, _QP), jnp.float32)
    fshape = jax.ShapeDtypeStruct((_N, _DOUT), jnp.float32)
    cshape = jax.ShapeDtypeStruct((_N, 1), jnp.float32)
    return pl.pallas_call(
        _prep_body,
        grid=(_GRID,),
        in_specs=[cspec, cspec, fspec],
        out_specs=[qspec, qspec, qspec, qspec, fspec, fspec, cspec, cspec],
        out_shape=[qshape, qshape, qshape, qshape, fshape, fshape, cshape,
                   cshape],
    )(d0, d1, h)


def _prop_body(uq0, uq1, uq2, uq3, src2, dst2, zeros2,
               o0, o1, o2, o3, sidx_v, didx_v, rows_v, zo_v, acc_sh, gsem):
    c = lax.axis_index("c")
    s = lax.axis_index("s")
    rbase = s * jnp.int32(_PT // _LANE)

    for p in range(2):
        tab_c0 = uq0 if p == 0 else uq1
        tab_c1 = uq2 if p == 0 else uq3
        out_c0 = o0 if p == 0 else o1
        out_c1 = o2 if p == 0 else o3

        pltpu.sync_copy(zeros2, zo_v)
        for k in range(_RPT // _ZB):
            pltpu.sync_copy(zo_v, acc_sh.at[pl.ds(s * _RPT + k * _ZB, _ZB)])
        plsc.subcore_barrier()

        @pl.loop(jnp.int32(0), jnp.int32(_NCHUNK))
        def _(i):
            r = rbase + i * jnp.int32(_CROWS)
            pltpu.sync_copy(src2.at[pl.ds(r, _CROWS)], sidx_v)
            pltpu.sync_copy(dst2.at[pl.ds(r, _CROWS)], didx_v)

            @pl.when(c == 0)
            def _():
                for j in range(_CROWS):
                    pltpu.async_copy(tab_c0.at[sidx_v.at[jnp.int32(j)]],
                                     rows_v.at[jnp.int32(j)], gsem)

            @pl.when(c == 1)
            def _():
                for j in range(_CROWS):
                    pltpu.async_copy(tab_c1.at[sidx_v.at[jnp.int32(j)]],
                                     rows_v.at[jnp.int32(j)], gsem)

            for j in range(_CROWS):
                pltpu.make_async_copy(tab_c0.at[sidx_v.at[jnp.int32(j)]],
                                      rows_v.at[jnp.int32(j)], gsem).wait()
            for j in range(_CROWS):
                pltpu.sync_copy(rows_v.at[jnp.int32(j)],
                                acc_sh.at[didx_v.at[jnp.int32(j)]], add=True)
        plsc.subcore_barrier()

        for k in range(_RPT // _ZB):
            pltpu.sync_copy(acc_sh.at[pl.ds(s * _RPT + k * _ZB, _ZB)], zo_v)

            @pl.when(c == 0)
            def _():
                pltpu.sync_copy(zo_v,
                                out_c0.at[pl.ds(s * _RPT + k * _ZB, _ZB)])

            @pl.when(c == 1)
            def _():
                pltpu.sync_copy(zo_v,
                                out_c1.at[pl.ds(s * _RPT + k * _ZB, _ZB)])


def _propagate(uq0, uq1, uq2, uq3, src2, dst2, zeros2):
    oshape = jax.ShapeDtypeStruct((_NR, _QP), jnp.float32)
    kern = pl.kernel(
        _prop_body,
        out_type=(oshape, oshape, oshape, oshape),
        mesh=_mesh(),
        scratch_types=[
            pltpu.VMEM((_CROWS, _LANE), jnp.int32),
            pltpu.VMEM((_CROWS, _LANE), jnp.int32),
            pltpu.VMEM((_CROWS, _LANE, _QP), jnp.float32),
            pltpu.VMEM((_ZB, _QP), jnp.float32),
            pltpu.VMEM_SHARED((_NR, _QP), jnp.float32),
            pltpu.SemaphoreType.DMA,
        ],
        compiler_params=pltpu.CompilerParams(use_tc_tiling_on_sc=False),
    )
    return kern(uq0, uq1, uq2, uq3, src2, dst2, zeros2)


def _mid_body(a0_ref, a1_ref, a2_ref, a3_ref, sd_ref, dbh_ref,
              u0_ref, u1_ref, u2_ref, u3_ref):
    sd = sd_ref[...]
    dbh = dbh_ref[...]
    zp = jnp.zeros((sd.shape[0], _QP - _QC), jnp.float32)
    u0_ref[...] = jnp.concatenate(
        [sd * a0_ref[...][:, :_QC] + dbh[:, 0 * _QC:1 * _QC], zp], axis=1)
    u1_ref[...] = jnp.concatenate(
        [sd * a1_ref[...][:, :_QC] + dbh[:, 1 * _QC:2 * _QC], zp], axis=1)
    u2_ref[...] = jnp.concatenate(
        [sd * a2_ref[...][:, :_QC] + dbh[:, 2 * _QC:3 * _QC], zp], axis=1)
    u3_ref[...] = jnp.concatenate(
        [sd * a3_ref[...][:, :_QC] + dbh[:, 3 * _QC:4 * _QC], zp], axis=1)


def _mid(a0, a1, a2, a3, sd, dbh):
    qspec = pl.BlockSpec((_BR, _QP), lambda i: (i, _Z))
    fspec = pl.BlockSpec((_BR, _DOUT), lambda i: (i, _Z))
    cspec = pl.BlockSpec((_BR, 1), lambda i: (i, _Z))
    qshape = jax.ShapeDtypeStruct((_N, _QP), jnp.float32)
    return pl.pallas_call(
        _mid_body,
        grid=(_GRID,),
        in_specs=[qspec, qspec, qspec, qspec, cspec, fspec],
        out_specs=[qspec, qspec, qspec, qspec],
        out_shape=[qshape, qshape, qshape, qshape],
    )(a0, a1, a2, a3, sd, dbh)


def _final_body(a0_ref, a1_ref, a2_ref, a3_ref, s_ref, bh_ref, out_ref):
    sc = s_ref[...]
    bh = bh_ref[...]
    f = jnp.concatenate(
        [sc * a0_ref[...][:, :_QC] + bh[:, 0 * _QC:1 * _QC],
         sc * a1_ref[...][:, :_QC] + bh[:, 1 * _QC:2 * _QC],
         sc * a2_ref[...][:, :_QC] + bh[:, 2 * _QC:3 * _QC],
         sc * a3_ref[...][:, :_QC] + bh[:, 3 * _QC:4 * _QC]], axis=1)
    m = jnp.max(f, axis=1, keepdims=True)
    z = f - m
    out_ref[...] = z - jnp.log(jnp.sum(jnp.exp(z), axis=1, keepdims=True))


def _final(a0, a1, a2, a3, s, bh):
    qspec = pl.BlockSpec((_BR, _QP), lambda i: (i, _Z))
    fspec = pl.BlockSpec((_BR, _DOUT), lambda i: (i, _Z))
    cspec = pl.BlockSpec((_BR, 1), lambda i: (i, _Z))
    return pl.pallas_call(
        _final_body,
        grid=(_GRID,),
        in_specs=[qspec, qspec, qspec, qspec, cspec, fspec],
        out_specs=fspec,
        out_shape=jax.ShapeDtypeStruct((_N, _DOUT), jnp.float32),
    )(a0, a1, a2, a3, s, bh)


@jax.jit
def _run(x, edge_index, W1, b1, Wc, bc):
    out_dtype = jnp.promote_types(jnp.promote_types(x.dtype, W1.dtype),
                                  jnp.promote_types(Wc.dtype, jnp.float32))
    x = x.astype(jnp.float32)
    w1t = W1.astype(jnp.float32).T
    wct = Wc.astype(jnp.float32).T
    b1r = b1.astype(jnp.float32).reshape(1, 16)
    bcr = bc.astype(jnp.float32).reshape(1, _DOUT)

    src = edge_index[0].astype(jnp.int32)
    dst = edge_index[1].astype(jnp.int32)
    npad = _EPAD - _E
    pad_ar = lax.iota(jnp.int32, npad)
    src_pad = jnp.concatenate([src, (pad_ar * 9973) % _N])
    dst_pad = jnp.concatenate([dst, _N + (pad_ar % _PADR)])
    src2 = src_pad.reshape(_EROWS, _LANE)
    dst2 = dst_pad.reshape(_EROWS, _LANE)

    zflat = jnp.zeros((_ZD,), jnp.float32)
    zeros2 = jnp.zeros((_ZB, _QP), jnp.float32)

    h = _compute_h(x, w1t, b1r, wct, bcr)
    dd = _compute_deg(dst2, zflat)
    d0 = dd[0, :_N].reshape(_N, 1)
    d1 = dd[1, :_N].reshape(_N, 1)

    u10, u11, u12, u13, bh, dbh, s, sd = _prepare(d0, d1, h)

    a0, a1, a2, a3 = _propagate(u10, u11, u12, u13, src2, dst2, zeros2)
    u20, u21, u22, u23 = _mid(a0[:_N], a1[:_N], a2[:_N], a3[:_N], sd, dbh)

    b0, b1_, b2, b3 = _propagate(u20, u21, u22, u23, src2, dst2, zeros2)
    out = _final(b0[:_N], b1_[:_N], b2[:_N], b3[:_N], s, bh)
    return out.astype(out_dtype)


def kernel(x, edge_index, W1, b1, Wc, bc):
    return _run(x, edge_index, W1, b1, Wc, bc)


# speedup vs baseline: 263.6129x; 1.1341x over previous
import jax
import jax.numpy as jnp
import numpy as np
from jax import lax
from jax.experimental import pallas as pl
from jax.experimental.pallas import tpu as pltpu
from jax.experimental.pallas import tpu_sc as plsc

_N = 100000
_E = 1600000
_DOUT = 40
_QC = 10
_QP = 16
_MU = 0.1
_EPS = 1e-8

_LANE = 128
_CROWS = 8
_CHUNK = _LANE * _CROWS

_PT = 100352
_NCHUNK = _PT // _CHUNK
_EPAD = 16 * _PT
_EROWS = _EPAD // _LANE

_PT_DEG = _EPAD // 32
_NCHUNK_DEG = _PT_DEG // _CHUNK

_PADR = 96
_NR = 100096
_RPT = _NR // 16

_ZD = 368
_ZB = 368

_BR = 2000
_GRID = _N // _BR

_ALPHA_POS = 1.0 / (1.0 + _MU)
_Z = np.int32(0)


def _mesh():
    return plsc.VectorSubcoreMesh(core_axis_name="c", subcore_axis_name="s")


def _mm_body(x_ref, w1t_ref, b1_ref, wct_ref, bc_ref, out_ref):
    h = jnp.maximum(
        jnp.dot(x_ref[...], w1t_ref[...], preferred_element_type=jnp.float32)
        + b1_ref[...], 0.0)
    out_ref[...] = (
        jnp.dot(h, wct_ref[...], preferred_element_type=jnp.float32)
        + bc_ref[...])


def _compute_h(x, w1t, b1, wct, bc):
    return pl.pallas_call(
        _mm_body,
        grid=(_GRID,),
        in_specs=[
            pl.BlockSpec((_BR, 128), lambda i: (i, _Z)),
            pl.BlockSpec((128, 16), lambda i: (_Z, _Z)),
            pl.BlockSpec((1, 16), lambda i: (_Z, _Z)),
            pl.BlockSpec((16, _DOUT), lambda i: (_Z, _Z)),
            pl.BlockSpec((1, _DOUT), lambda i: (_Z, _Z)),
        ],
        out_specs=pl.BlockSpec((_BR, _DOUT), lambda i: (i, _Z)),
        out_shape=jax.ShapeDtypeStruct((_N, _DOUT), jnp.float32),
    )(x, w1t, b1, wct, bc)


def _deg_body(dst2, zflat, d_out, didx_v, ones_v, zd_v, deg_sh, sem):
    del sem
    c = lax.axis_index("c")
    s = lax.axis_index("s")
    w = c * 16 + s
    pltpu.sync_copy(zflat, zd_v)
    for k in range(_RPT // _ZD):
        pltpu.sync_copy(zd_v, deg_sh.at[pl.ds(s * _RPT + k * _ZD, _ZD)])
    for i in range(8):
        ones_v[pl.ds(i * 16, 16)] = jnp.ones((16,), jnp.float32)
    plsc.subcore_barrier()

    rbase = w * jnp.int32(_PT_DEG // _LANE)

    @pl.loop(jnp.int32(0), jnp.int32(_NCHUNK_DEG))
    def _(i):
        r = rbase + i * jnp.int32(_CROWS)
        pltpu.sync_copy(dst2.at[pl.ds(r, _CROWS)], didx_v)
        for j in range(_CROWS):
            pltpu.sync_copy(ones_v, deg_sh.at[didx_v.at[jnp.int32(j)]],
                            add=True)
    plsc.subcore_barrier()

    for k in range(_RPT // _ZD):
        pltpu.sync_copy(deg_sh.at[pl.ds(s * _RPT + k * _ZD, _ZD)], zd_v)
        pltpu.sync_copy(zd_v, d_out.at[c, pl.ds(s * _RPT + k * _ZD, _ZD)])


def _compute_deg(dst2, zflat):
    kern = pl.kernel(
        _deg_body,
        out_type=jax.ShapeDtypeStruct((2, _NR), jnp.float32),
        mesh=_mesh(),
        scratch_types=[
            pltpu.VMEM((_CROWS, _LANE), jnp.int32),
            pltpu.VMEM((_LANE,), jnp.float32),
            pltpu.VMEM((_ZD,), jnp.float32),
            pltpu.VMEM_SHARED((_NR,), jnp.float32),
            pltpu.SemaphoreType.DMA,
        ],
        compiler_params=pltpu.CompilerParams(use_tc_tiling_on_sc=False),
    )
    return kern(dst2, zflat)


def _prep_body(d0_ref, d1_ref, h_ref, uq0_ref, uq1_ref, uq2_ref, uq3_ref,
               bh_ref, dbh_ref, s_ref, sd_ref):
    deg = d0_ref[...] + d1_ref[...]
    pos = deg > 0
    dinv = jnp.where(pos, lax.rsqrt(jnp.maximum(deg, _EPS)), 0.0)
    alpha = jnp.where(pos, jnp.float32(_ALPHA_POS), jnp.float32(1.0 / _MU))
    beta = _MU * alpha
    h = h_ref[...]
    u1 = h * dinv
    zp = jnp.zeros((u1.shape[0], _QP - _QC), jnp.float32)
    uq0_ref[...] = jnp.concatenate([u1[:, 0 * _QC:1 * _QC], zp], axis=1)
    uq1_ref[...] = jnp.concatenate([u1[:, 1 * _QC:2 * _QC], zp], axis=1)
    uq2_ref[...] = jnp.concatenate([u1[:, 2 * _QC:3 * _QC], zp], axis=1)
    uq3_ref[...] = jnp.concatenate([u1[:, 3 * _QC:4 * _QC], zp], axis=1)
    bh = beta * h
    bh_ref[...] = bh
    dbh_ref[...] = dinv * bh
    s_ref[...] = alpha * dinv
    sd_ref[...] = alpha * dinv * dinv


def _prepare(d0, d1, h):
    qspec = pl.BlockSpec((_BR, _QP), lambda i: (i, _Z))
    fspec = pl.BlockSpec((_BR, _DOUT), lambda i: (i, _Z))
    cspec = pl.BlockSpec((_BR, 1), lambda i: (i, _Z))
    qshape = jax.ShapeDtypeStruct((_N, _QP), jnp.float32)
    fshape = jax.ShapeDtypeStruct((_N, _DOUT), jnp.float32)
    cshape = jax.ShapeDtypeStruct((_N, 1), jnp.float32)
    return pl.pallas_call(
        _prep_body,
        grid=(_GRID,),
        in_specs=[cspec, cspec, fspec],
        out_specs=[qspec, qspec, qspec, qspec, fspec, fspec, cspec, cspec],
        out_shape=[qshape, qshape, qshape, qshape, fshape, fshape, cshape,
                   cshape],
    )(d0, d1, h)


def _prop_body(uq0, uq1, uq2, uq3, src2, dst2, zeros2,
               o0, o1, o2, o3, sidx_v, didx_v, rows_v, zo_v, acc_sh, gsem):
    c = lax.axis_index("c")
    s = lax.axis_index("s")
    rbase = s * jnp.int32(_PT // _LANE)

    for p in range(2):
        tab_c0 = uq0 if p == 0 else uq1
        tab_c1 = uq2 if p == 0 else uq3
        out_c0 = o0 if p == 0 else o1
        out_c1 = o2 if p == 0 else o3

        pltpu.sync_copy(zeros2, zo_v)
        for k in range(_RPT // _ZB):
            pltpu.sync_copy(zo_v, acc_sh.at[pl.ds(s * _RPT + k * _ZB, _ZB)])
        plsc.subcore_barrier()

        @pl.loop(jnp.int32(0), jnp.int32(_NCHUNK))
        def _(i):
            r = rbase + i * jnp.int32(_CROWS)
            pltpu.sync_copy(src2.at[pl.ds(r, _CROWS)], sidx_v)
            pltpu.sync_copy(dst2.at[pl.ds(r, _CROWS)], didx_v)

            @pl.when(c == 0)
            def _():
                for j in range(_CROWS):
                    pltpu.async_copy(tab_c0.at[sidx_v.at[jnp.int32(j)]],
                                     rows_v.at[jnp.int32(j)], gsem)

            @pl.when(c == 1)
            def _():
                for j in range(_CROWS):
                    pltpu.async_copy(tab_c1.at[sidx_v.at[jnp.int32(j)]],
                                     rows_v.at[jnp.int32(j)], gsem)

            for j in range(_CROWS):
                pltpu.make_async_copy(tab_c0.at[sidx_v.at[jnp.int32(j)]],
                                      rows_v.at[jnp.int32(j)], gsem).wait()
            for j in range(_CROWS):
                pltpu.sync_copy(rows_v.at[jnp.int32(j)],
                                acc_sh.at[didx_v.at[jnp.int32(j)]], add=True)
        plsc.subcore_barrier()

        for k in range(_RPT // _ZB):
            pltpu.sync_copy(acc_sh.at[pl.ds(s * _RPT + k * _ZB, _ZB)], zo_v)

            @pl.when(c == 0)
            def _():
                pltpu.sync_copy(zo_v,
                                out_c0.at[pl.ds(s * _RPT + k * _ZB, _ZB)])

            @pl.when(c == 1)
            def _():
                pltpu.sync_copy(zo_v,
                                out_c1.at[pl.ds(s * _RPT + k * _ZB, _ZB)])


def _propagate(uq0, uq1, uq2, uq3, src2, dst2, zeros2):
    oshape = jax.ShapeDtypeStruct((_NR, _QP), jnp.float32)
    kern = pl.kernel(
        _prop_body,
        out_type=(oshape, oshape, oshape, oshape),
        mesh=_mesh(),
        scratch_types=[
            pltpu.VMEM((_CROWS, _LANE), jnp.int32),
            pltpu.VMEM((_CROWS, _LANE), jnp.int32),
            pltpu.VMEM((_CROWS, _LANE, _QP), jnp.float32),
            pltpu.VMEM((_ZB, _QP), jnp.float32),
            pltpu.VMEM_SHARED((_NR, _QP), jnp.float32),
            pltpu.SemaphoreType.DMA,
        ],
        compiler_params=pltpu.CompilerParams(use_tc_tiling_on_sc=False),
    )
    return kern(uq0, uq1, uq2, uq3, src2, dst2, zeros2)


def _mid_body(a0_ref, a1_ref, a2_ref, a3_ref, sd_ref, dbh_ref,
              u0_ref, u1_ref, u2_ref, u3_ref):
    sd = sd_ref[...]
    dbh = dbh_ref[...]
    zp = jnp.zeros((sd.shape[0], _QP - _QC), jnp.float32)
    u0_ref[...] = jnp.concatenate(
        [sd * a0_ref[...][:, :_QC] + dbh[:, 0 * _QC:1 * _QC], zp], axis=1)
    u1_ref[...] = jnp.concatenate(
        [sd * a1_ref[...][:, :_QC] + dbh[:, 1 * _QC:2 * _QC], zp], axis=1)
    u2_ref[...] = jnp.concatenate(
        [sd * a2_ref[...][:, :_QC] + dbh[:, 2 * _QC:3 * _QC], zp], axis=1)
    u3_ref[...] = jnp.concatenate(
        [sd * a3_ref[...][:, :_QC] + dbh[:, 3 * _QC:4 * _QC], zp], axis=1)


def _mid(a0, a1, a2, a3, sd, dbh):
    qspec = pl.BlockSpec((_BR, _QP), lambda i: (i, _Z))
    fspec = pl.BlockSpec((_BR, _DOUT), lambda i: (i, _Z))
    cspec = pl.BlockSpec((_BR, 1), lambda i: (i, _Z))
    qshape = jax.ShapeDtypeStruct((_N, _QP), jnp.float32)
    return pl.pallas_call(
        _mid_body,
        grid=(_GRID,),
        in_specs=[qspec, qspec, qspec, qspec, cspec, fspec],
        out_specs=[qspec, qspec, qspec, qspec],
        out_shape=[qshape, qshape, qshape, qshape],
    )(a0, a1, a2, a3, sd, dbh)


def _final_body(a0_ref, a1_ref, a2_ref, a3_ref, s_ref, bh_ref, hi_ref,
                lo_ref):
    sc = s_ref[...]
    bh = bh_ref[...]
    f = jnp.concatenate(
        [sc * a0_ref[...][:, :_QC] + bh[:, 0 * _QC:1 * _QC],
         sc * a1_ref[...][:, :_QC] + bh[:, 1 * _QC:2 * _QC],
         sc * a2_ref[...][:, :_QC] + bh[:, 2 * _QC:3 * _QC],
         sc * a3_ref[...][:, :_QC] + bh[:, 3 * _QC:4 * _QC]], axis=1)
    m = jnp.max(f, axis=1, keepdims=True)
    z = f - m
    out = z - jnp.log(jnp.sum(jnp.exp(z), axis=1, keepdims=True))
    bu = lax.bitcast_convert_type(out, jnp.uint32)
    e = (bu >> 23) & jnp.uint32(0xFF)
    mant = bu & jnp.uint32(0x7FFFFF)
    eb = jnp.where(e == 255, jnp.uint32(2047), e + jnp.uint32(896))
    hi = (bu & jnp.uint32(0x80000000)) | (eb << 20) | (mant >> 3)
    lo = mant << 29
    zd = e == 0
    hi_ref[...] = jnp.where(zd, bu & jnp.uint32(0x80000000), hi)
    lo_ref[...] = jnp.where(zd, jnp.uint32(0), lo)


def _final(a0, a1, a2, a3, s, bh):
    qspec = pl.BlockSpec((_BR, _QP), lambda i: (i, _Z))
    fspec = pl.BlockSpec((_BR, _DOUT), lambda i: (i, _Z))
    cspec = pl.BlockSpec((_BR, 1), lambda i: (i, _Z))
    return pl.pallas_call(
        _final_body,
        grid=(_GRID,),
        in_specs=[qspec, qspec, qspec, qspec, cspec, fspec],
        out_specs=[fspec, fspec],
        out_shape=[jax.ShapeDtypeStruct((_N, _DOUT), jnp.uint32),
                   jax.ShapeDtypeStruct((_N, _DOUT), jnp.uint32)],
    )(a0, a1, a2, a3, s, bh)


@jax.jit
def _run(x, edge_index, W1, b1, Wc, bc):
    out_dtype = jnp.promote_types(jnp.promote_types(x.dtype, W1.dtype),
                                  jnp.promote_types(Wc.dtype, jnp.float32))
    x = x.astype(jnp.float32)
    w1t = W1.astype(jnp.float32).T
    wct = Wc.astype(jnp.float32).T
    b1r = b1.astype(jnp.float32).reshape(1, 16)
    bcr = bc.astype(jnp.float32).reshape(1, _DOUT)

    src = edge_index[0].astype(jnp.int32)
    dst = edge_index[1].astype(jnp.int32)
    npad = _EPAD - _E
    pad_ar = lax.iota(jnp.int32, npad)
    src_pad = jnp.concatenate([src, (pad_ar * 9973) % _N])
    dst_pad = jnp.concatenate([dst, _N + (pad_ar % _PADR)])
    src2 = src_pad.reshape(_EROWS, _LANE)
    dst2 = dst_pad.reshape(_EROWS, _LANE)

    zflat = jnp.zeros((_ZD,), jnp.float32)
    zeros2 = jnp.zeros((_ZB, _QP), jnp.float32)

    h = _compute_h(x, w1t, b1r, wct, bcr)
    dd = _compute_deg(dst2, zflat)
    d0 = dd[0, :_N].reshape(_N, 1)
    d1 = dd[1, :_N].reshape(_N, 1)

    u10, u11, u12, u13, bh, dbh, s, sd = _prepare(d0, d1, h)

    a0, a1, a2, a3 = _propagate(u10, u11, u12, u13, src2, dst2, zeros2)
    u20, u21, u22, u23 = _mid(a0[:_N], a1[:_N], a2[:_N], a3[:_N], sd, dbh)

    b0, b1_, b2, b3 = _propagate(u20, u21, u22, u23, src2, dst2, zeros2)
    hi, lo = _final(b0[:_N], b1_[:_N], b2[:_N], b3[:_N], s, bh)
    if out_dtype == jnp.float64:
        pairs = jnp.stack([lo, hi], axis=-1)
        return lax.bitcast_convert_type(pairs, jnp.float64)
    sign = hi & jnp.uint32(0x80000000)
    eb = (hi >> 20) & jnp.uint32(0x7FF)
    mant = ((hi & jnp.uint32(0xFFFFF)) << 3) | (lo >> 29)
    b32 = jnp.where(eb == 0, sign, sign | ((eb - 896) << 23) | mant)
    return lax.bitcast_convert_type(b32, jnp.float32).astype(out_dtype)


def kernel(x, edge_index, W1, b1, Wc, bc):
    return _run(x, edge_index, W1, b1, Wc, bc)


# speedup vs baseline: 284.0553x; 1.0775x over previous
import jax
import jax.numpy as jnp
import numpy as np
from jax import lax
from jax.experimental import pallas as pl
from jax.experimental.pallas import tpu as pltpu
from jax.experimental.pallas import tpu_sc as plsc

_N = 100000
_E = 1600000
_DOUT = 40
_QC = 10
_QP = 16
_MU = 0.1
_EPS = 1e-8

_LANE = 128
_CROWS = 4
_CHUNK = _LANE * _CROWS

_PT = 100352
_NCHUNK = _PT // _CHUNK
_EPAD = 16 * _PT
_EROWS = _EPAD // _LANE

_PT_DEG = _EPAD // 32
_NCHUNK_DEG = _PT_DEG // _CHUNK

_PADR = 96
_NR = 100096
_RPT = _NR // 16

_ZD = 368
_ZB = 368

_BR = 2000
_GRID = _N // _BR

_ALPHA_POS = 1.0 / (1.0 + _MU)
_Z = np.int32(0)


def _mesh():
    return plsc.VectorSubcoreMesh(core_axis_name="c", subcore_axis_name="s")


def _mm_body(x_ref, w1t_ref, b1_ref, wct_ref, bc_ref, out_ref):
    h = jnp.maximum(
        jnp.dot(x_ref[...], w1t_ref[...], preferred_element_type=jnp.float32)
        + b1_ref[...], 0.0)
    out_ref[...] = (
        jnp.dot(h, wct_ref[...], preferred_element_type=jnp.float32)
        + bc_ref[...])


def _compute_h(x, w1t, b1, wct, bc):
    return pl.pallas_call(
        _mm_body,
        grid=(_GRID,),
        in_specs=[
            pl.BlockSpec((_BR, 128), lambda i: (i, _Z)),
            pl.BlockSpec((128, 16), lambda i: (_Z, _Z)),
            pl.BlockSpec((1, 16), lambda i: (_Z, _Z)),
            pl.BlockSpec((16, _DOUT), lambda i: (_Z, _Z)),
            pl.BlockSpec((1, _DOUT), lambda i: (_Z, _Z)),
        ],
        out_specs=pl.BlockSpec((_BR, _DOUT), lambda i: (i, _Z)),
        out_shape=jax.ShapeDtypeStruct((_N, _DOUT), jnp.float32),
    )(x, w1t, b1, wct, bc)


def _deg_body(dst2, zflat, d_out, didx_v, ones_v, zd_v, deg_sh, sem):
    del sem
    c = lax.axis_index("c")
    s = lax.axis_index("s")
    w = c * 16 + s
    pltpu.sync_copy(zflat, zd_v)
    for k in range(_RPT // _ZD):
        pltpu.sync_copy(zd_v, deg_sh.at[pl.ds(s * _RPT + k * _ZD, _ZD)])
    for i in range(8):
        ones_v[pl.ds(i * 16, 16)] = jnp.ones((16,), jnp.float32)
    plsc.subcore_barrier()

    rbase = w * jnp.int32(_PT_DEG // _LANE)

    @pl.loop(jnp.int32(0), jnp.int32(_NCHUNK_DEG))
    def _(i):
        r = rbase + i * jnp.int32(_CROWS)
        pltpu.sync_copy(dst2.at[pl.ds(r, _CROWS)], didx_v)
        for j in range(_CROWS):
            pltpu.sync_copy(ones_v, deg_sh.at[didx_v.at[jnp.int32(j)]],
                            add=True)
    plsc.subcore_barrier()

    for k in range(_RPT // _ZD):
        pltpu.sync_copy(deg_sh.at[pl.ds(s * _RPT + k * _ZD, _ZD)], zd_v)
        pltpu.sync_copy(zd_v, d_out.at[c, pl.ds(s * _RPT + k * _ZD, _ZD)])


def _compute_deg(dst2, zflat):
    kern = pl.kernel(
        _deg_body,
        out_type=jax.ShapeDtypeStruct((2, _NR), jnp.float32),
        mesh=_mesh(),
        scratch_types=[
            pltpu.VMEM((_CROWS, _LANE), jnp.int32),
            pltpu.VMEM((_LANE,), jnp.float32),
            pltpu.VMEM((_ZD,), jnp.float32),
            pltpu.VMEM_SHARED((_NR,), jnp.float32),
            pltpu.SemaphoreType.DMA,
        ],
        compiler_params=pltpu.CompilerParams(use_tc_tiling_on_sc=False),
    )
    return kern(dst2, zflat)


def _prep_body(d0_ref, d1_ref, h_ref, uq0_ref, uq1_ref, uq2_ref, uq3_ref,
               bh_ref, dbh_ref, s_ref, sd_ref):
    deg = d0_ref[...] + d1_ref[...]
    pos = deg > 0
    dinv = jnp.where(pos, lax.rsqrt(jnp.maximum(deg, _EPS)), 0.0)
    alpha = jnp.where(pos, jnp.float32(_ALPHA_POS), jnp.float32(1.0 / _MU))
    beta = _MU * alpha
    h = h_ref[...]
    u1 = h * dinv
    zp = jnp.zeros((u1.shape[0], _QP - _QC), jnp.float32)
    uq0_ref[...] = jnp.concatenate([u1[:, 0 * _QC:1 * _QC], zp], axis=1)
    uq1_ref[...] = jnp.concatenate([u1[:, 1 * _QC:2 * _QC], zp], axis=1)
    uq2_ref[...] = jnp.concatenate([u1[:, 2 * _QC:3 * _QC], zp], axis=1)
    uq3_ref[...] = jnp.concatenate([u1[:, 3 * _QC:4 * _QC], zp], axis=1)
    bh = beta * h
    bh_ref[...] = bh
    dbh_ref[...] = dinv * bh
    s_ref[...] = alpha * dinv
    sd_ref[...] = alpha * dinv * dinv


def _prepare(d0, d1, h):
    qspec = pl.BlockSpec((_BR, _QP), lambda i: (i, _Z))
    fspec = pl.BlockSpec((_BR, _DOUT), lambda i: (i, _Z))
    cspec = pl.BlockSpec((_BR, 1), lambda i: (i, _Z))
    qshape = jax.ShapeDtypeStruct((_N, _QP), jnp.float32)
    fshape = jax.ShapeDtypeStruct((_N, _DOUT), jnp.float32)
    cshape = jax.ShapeDtypeStruct((_N, 1), jnp.float32)
    return pl.pallas_call(
        _prep_body,
        grid=(_GRID,),
        in_specs=[cspec, cspec, fspec],
        out_specs=[qspec, qspec, qspec, qspec, fspec, fspec, cspec, cspec],
        out_shape=[qshape, qshape, qshape, qshape, fshape, fshape, cshape,
                   cshape],
    )(d0, d1, h)


def _prop_body(uq0, uq1, uq2, uq3, src2, dst2, zeros2,
               o0, o1, o2, o3, sidx_v, didx_v, rows_v, zo_v, acc_sh,
               sem0, sem1):
    c = lax.axis_index("c")
    s = lax.axis_index("s")
    rbase = s * jnp.int32(_PT // _LANE)
    sems = (sem0, sem1)
    npair = _NCHUNK // 2

    for p in range(2):
        tab_c0 = uq0 if p == 0 else uq1
        tab_c1 = uq2 if p == 0 else uq3
        out_c0 = o0 if p == 0 else o1
        out_c1 = o2 if p == 0 else o3

        def fire(i, b):
            r = rbase + i * jnp.int32(_CROWS)
            pltpu.sync_copy(src2.at[pl.ds(r, _CROWS)], sidx_v.at[jnp.int32(b)])
            pltpu.sync_copy(dst2.at[pl.ds(r, _CROWS)], didx_v.at[jnp.int32(b)])

            @pl.when(c == 0)
            def _():
                for j in range(_CROWS):
                    pltpu.async_copy(
                        tab_c0.at[sidx_v.at[jnp.int32(b)].at[jnp.int32(j)]],
                        rows_v.at[jnp.int32(b)].at[jnp.int32(j)], sems[b])

            @pl.when(c == 1)
            def _():
                for j in range(_CROWS):
                    pltpu.async_copy(
                        tab_c1.at[sidx_v.at[jnp.int32(b)].at[jnp.int32(j)]],
                        rows_v.at[jnp.int32(b)].at[jnp.int32(j)], sems[b])

        def drain_scatter(b):
            for j in range(_CROWS):
                pltpu.make_async_copy(
                    tab_c0.at[sidx_v.at[jnp.int32(b)].at[jnp.int32(j)]],
                    rows_v.at[jnp.int32(b)].at[jnp.int32(j)], sems[b]).wait()
            for j in range(_CROWS):
                pltpu.sync_copy(
                    rows_v.at[jnp.int32(b)].at[jnp.int32(j)],
                    acc_sh.at[didx_v.at[jnp.int32(b)].at[jnp.int32(j)]],
                    add=True)

        pltpu.sync_copy(zeros2, zo_v)
        for k in range(_RPT // _ZB):
            pltpu.sync_copy(zo_v, acc_sh.at[pl.ds(s * _RPT + k * _ZB, _ZB)])
        plsc.subcore_barrier()

        fire(jnp.int32(0), 0)

        @pl.loop(jnp.int32(0), jnp.int32(npair))
        def _(k):
            i = k * jnp.int32(2)
            fire(i + jnp.int32(1), 1)
            drain_scatter(0)

            @pl.when(k < jnp.int32(npair - 1))
            def _():
                fire(i + jnp.int32(2), 0)

            drain_scatter(1)

        plsc.subcore_barrier()

        for k in range(_RPT // _ZB):
            pltpu.sync_copy(acc_sh.at[pl.ds(s * _RPT + k * _ZB, _ZB)], zo_v)

            @pl.when(c == 0)
            def _():
                pltpu.sync_copy(zo_v,
                                out_c0.at[pl.ds(s * _RPT + k * _ZB, _ZB)])

            @pl.when(c == 1)
            def _():
                pltpu.sync_copy(zo_v,
                                out_c1.at[pl.ds(s * _RPT + k * _ZB, _ZB)])


def _propagate(uq0, uq1, uq2, uq3, src2, dst2, zeros2):
    oshape = jax.ShapeDtypeStruct((_NR, _QP), jnp.float32)
    kern = pl.kernel(
        _prop_body,
        out_type=(oshape, oshape, oshape, oshape),
        mesh=_mesh(),
        scratch_types=[
            pltpu.VMEM((2, _CROWS, _LANE), jnp.int32),
            pltpu.VMEM((2, _CROWS, _LANE), jnp.int32),
            pltpu.VMEM((2, _CROWS, _LANE, _QP), jnp.float32),
            pltpu.VMEM((_ZB, _QP), jnp.float32),
            pltpu.VMEM_SHARED((_NR, _QP), jnp.float32),
            pltpu.SemaphoreType.DMA,
            pltpu.SemaphoreType.DMA,
        ],
        compiler_params=pltpu.CompilerParams(use_tc_tiling_on_sc=False),
    )
    return kern(uq0, uq1, uq2, uq3, src2, dst2, zeros2)


def _mid_body(a0_ref, a1_ref, a2_ref, a3_ref, sd_ref, dbh_ref,
              u0_ref, u1_ref, u2_ref, u3_ref):
    sd = sd_ref[...]
    dbh = dbh_ref[...]
    zp = jnp.zeros((sd.shape[0], _QP - _QC), jnp.float32)
    u0_ref[...] = jnp.concatenate(
        [sd * a0_ref[...][:, :_QC] + dbh[:, 0 * _QC:1 * _QC], zp], axis=1)
    u1_ref[...] = jnp.concatenate(
        [sd * a1_ref[...][:, :_QC] + dbh[:, 1 * _QC:2 * _QC], zp], axis=1)
    u2_ref[...] = jnp.concatenate(
        [sd * a2_ref[...][:, :_QC] + dbh[:, 2 * _QC:3 * _QC], zp], axis=1)
    u3_ref[...] = jnp.concatenate(
        [sd * a3_ref[...][:, :_QC] + dbh[:, 3 * _QC:4 * _QC], zp], axis=1)


def _mid(a0, a1, a2, a3, sd, dbh):
    qspec = pl.BlockSpec((_BR, _QP), lambda i: (i, _Z))
    fspec = pl.BlockSpec((_BR, _DOUT), lambda i: (i, _Z))
    cspec = pl.BlockSpec((_BR, 1), lambda i: (i, _Z))
    qshape = jax.ShapeDtypeStruct((_N, _QP), jnp.float32)
    return pl.pallas_call(
        _mid_body,
        grid=(_GRID,),
        in_specs=[qspec, qspec, qspec, qspec, cspec, fspec],
        out_specs=[qspec, qspec, qspec, qspec],
        out_shape=[qshape, qshape, qshape, qshape],
    )(a0, a1, a2, a3, sd, dbh)


def _final_body(a0_ref, a1_ref, a2_ref, a3_ref, s_ref, bh_ref, hi_ref,
                lo_ref):
    sc = s_ref[...]
    bh = bh_ref[...]
    f = jnp.concatenate(
        [sc * a0_ref[...][:, :_QC] + bh[:, 0 * _QC:1 * _QC],
         sc * a1_ref[...][:, :_QC] + bh[:, 1 * _QC:2 * _QC],
         sc * a2_ref[...][:, :_QC] + bh[:, 2 * _QC:3 * _QC],
         sc * a3_ref[...][:, :_QC] + bh[:, 3 * _QC:4 * _QC]], axis=1)
    m = jnp.max(f, axis=1, keepdims=True)
    z = f - m
    out = z - jnp.log(jnp.sum(jnp.exp(z), axis=1, keepdims=True))
    bu = lax.bitcast_convert_type(out, jnp.uint32)
    e = (bu >> 23) & jnp.uint32(0xFF)
    mant = bu & jnp.uint32(0x7FFFFF)
    eb = jnp.where(e == 255, jnp.uint32(2047), e + jnp.uint32(896))
    hi = (bu & jnp.uint32(0x80000000)) | (eb << 20) | (mant >> 3)
    lo = mant << 29
    zd = e == 0
    hi_ref[...] = jnp.where(zd, bu & jnp.uint32(0x80000000), hi)
    lo_ref[...] = jnp.where(zd, jnp.uint32(0), lo)


def _final(a0, a1, a2, a3, s, bh):
    qspec = pl.BlockSpec((_BR, _QP), lambda i: (i, _Z))
    fspec = pl.BlockSpec((_BR, _DOUT), lambda i: (i, _Z))
    cspec = pl.BlockSpec((_BR, 1), lambda i: (i, _Z))
    return pl.pallas_call(
        _final_body,
        grid=(_GRID,),
        in_specs=[qspec, qspec, qspec, qspec, cspec, fspec],
        out_specs=[fspec, fspec],
        out_shape=[jax.ShapeDtypeStruct((_N, _DOUT), jnp.uint32),
                   jax.ShapeDtypeStruct((_N, _DOUT), jnp.uint32)],
    )(a0, a1, a2, a3, s, bh)


@jax.jit
def _run(x, edge_index, W1, b1, Wc, bc):
    out_dtype = jnp.promote_types(jnp.promote_types(x.dtype, W1.dtype),
                                  jnp.promote_types(Wc.dtype, jnp.float32))
    x = x.astype(jnp.float32)
    w1t = W1.astype(jnp.float32).T
    wct = Wc.astype(jnp.float32).T
    b1r = b1.astype(jnp.float32).reshape(1, 16)
    bcr = bc.astype(jnp.float32).reshape(1, _DOUT)

    src = edge_index[0].astype(jnp.int32)
    dst = edge_index[1].astype(jnp.int32)
    npad = _EPAD - _E
    pad_ar = lax.iota(jnp.int32, npad)
    src_pad = jnp.concatenate([src, (pad_ar * 9973) % _N])
    dst_pad = jnp.concatenate([dst, _N + (pad_ar % _PADR)])
    src2 = src_pad.reshape(_EROWS, _LANE)
    dst2 = dst_pad.reshape(_EROWS, _LANE)

    zflat = jnp.zeros((_ZD,), jnp.float32)
    zeros2 = jnp.zeros((_ZB, _QP), jnp.float32)

    h = _compute_h(x, w1t, b1r, wct, bcr)
    dd = _compute_deg(dst2, zflat)
    d0 = dd[0, :_N].reshape(_N, 1)
    d1 = dd[1, :_N].reshape(_N, 1)

    u10, u11, u12, u13, bh, dbh, s, sd = _prepare(d0, d1, h)

    a0, a1, a2, a3 = _propagate(u10, u11, u12, u13, src2, dst2, zeros2)
    u20, u21, u22, u23 = _mid(a0[:_N], a1[:_N], a2[:_N], a3[:_N], sd, dbh)

    b0, b1_, b2, b3 = _propagate(u20, u21, u22, u23, src2, dst2, zeros2)
    hi, lo = _final(b0[:_N], b1_[:_N], b2[:_N], b3[:_N], s, bh)
    if out_dtype == jnp.float64:
        pairs = jnp.stack([lo, hi], axis=-1)
        return lax.bitcast_convert_type(pairs, jnp.float64)
    sign = hi & jnp.uint32(0x80000000)
    eb = (hi >> 20) & jnp.uint32(0x7FF)
    mant = ((hi & jnp.uint32(0xFFFFF)) << 3) | (lo >> 29)
    b32 = jnp.where(eb == 0, sign, sign | ((eb - 896) << 23) | mant)
    return lax.bitcast_convert_type(b32, jnp.float32).astype(out_dtype)


def kernel(x, edge_index, W1, b1, Wc, bc):
    return _run(x, edge_index, W1, b1, Wc, bc)
